# baseline (speedup 1.0000x reference)
import functools

import jax
import jax.numpy as jnp
from jax import lax
from jax.experimental import pallas as pl
from jax.experimental.pallas import tpu as pltpu
from jax.experimental.pallas import tpu_sc as plsc

B = 16384
D = 64
K = 5
NC = 2
NS = 16
NW = NC * NS
N_PER_W = B // NW
CHUNK = 64
N_CHUNKS = N_PER_W // CHUNK
L = 16
GROUPS = CHUNK // L
HALF_ROWS = 1000000 // 2


def _make_sc_scores():
    mesh = plsc.VectorSubcoreMesh(core_axis_name="c", subcore_axis_name="s")
    scratch = (
        [pltpu.VMEM((CHUNK,), jnp.int32) for _ in range(2 + K)]
        + [pltpu.VMEM((CHUNK,), jnp.int32) for _ in range(2 + K)]
        + [pltpu.VMEM((CHUNK, 2 * D), jnp.float32) for _ in range(2 + K)]
        + [pltpu.VMEM((N_PER_W,), jnp.float32) for _ in range(1 + K)]
        + [pltpu.SemaphoreType.DMA]
    )

    @functools.partial(
        pl.kernel,
        out_type=[
            jax.ShapeDtypeStruct((B,), jnp.float32),
            jax.ShapeDtypeStruct((K * B,), jnp.float32),
        ],
        mesh=mesh,
        scratch_types=scratch,
        compiler_params=pltpu.CompilerParams(
            needs_layout_passes=False, use_tc_tiling_on_sc=True),
    )
    def sc_scores(center_hbm, context_hbm, neg_hbm, u_hbm, v_hbm,
                  pos_out, neg_out,
                  idx_c, idx_x, idx_n0, idx_n1, idx_n2, idx_n3, idx_n4,
                  hidx_c, hidx_x, hidx_n0, hidx_n1, hidx_n2, hidx_n3, hidx_n4,
                  c_rows, x_rows, n0, n1, n2, n3, n4,
                  pos_v, nv0, nv1, nv2, nv3, nv4, sem):
        idx_raw = [idx_c, idx_x, idx_n0, idx_n1, idx_n2, idx_n3, idx_n4]
        idx_h = [hidx_c, hidx_x, hidx_n0, hidx_n1, hidx_n2, hidx_n3, hidx_n4]
        rows = [c_rows, x_rows, n0, n1, n2, n3, n4]
        neg_v = [nv0, nv1, nv2, nv3, nv4]
        wid = lax.axis_index("s") * NC + lax.axis_index("c")
        base_w = wid * N_PER_W
        lanes = lax.iota(jnp.int32, L)

        def chunk_body(ci, _):
            base = base_w + ci * CHUNK
            srcs = [center_hbm, context_hbm] + [neg_hbm] * K
            offs = [base, base] + [kk * B + base for kk in range(K)]
            for t in range(2 + K):
                pltpu.sync_copy(srcs[t].at[pl.ds(offs[t], CHUNK)], idx_raw[t])
            for t in range(2 + K):
                for g in range(GROUPS):
                    sl = pl.ds(g * L, L)
                    idx_h[t][sl] = lax.shift_right_logical(idx_raw[t][sl], 1)
            copies = [
                pltpu.async_copy(u_hbm.at[idx_h[0]], c_rows, sem),
                pltpu.async_copy(v_hbm.at[idx_h[1]], x_rows, sem),
            ]
            for kk in range(K):
                copies.append(pltpu.async_copy(v_hbm.at[idx_h[2 + kk]], rows[2 + kk], sem))
            for cp in copies:
                cp.wait()

            def group_body(g, _, ci=ci):
                goff = g * L
                pvs = [(idx_raw[t][pl.ds(goff, L)] & 1) * D for t in range(2 + K)]
                scores = [jnp.zeros((L,), jnp.float32) for _ in range(1 + K)]
                for i in range(L):
                    e = goff + i
                    sel = lanes == i
                    c_off = pvs[0][i]
                    x_off = pvs[1][i]
                    c4 = [c_rows[e, pl.ds(c_off + q * L, L)] for q in range(D // L)]
                    x4 = [x_rows[e, pl.ds(x_off + q * L, L)] for q in range(D // L)]
                    acc = c4[0] * x4[0]
                    for q in range(1, D // L):
                        acc = acc + c4[q] * x4[q]
                    scores[0] = jnp.where(sel, jnp.sum(acc), scores[0])
                    for kk in range(K):
                        n_off = pvs[2 + kk][i]
                        n4 = [rows[2 + kk][e, pl.ds(n_off + q * L, L)]
                              for q in range(D // L)]
                        acc = c4[0] * n4[0]
                        for q in range(1, D // L):
                            acc = acc + c4[q] * n4[q]
                        scores[1 + kk] = jnp.where(sel, jnp.sum(acc), scores[1 + kk])
                off = ci * CHUNK + goff
                pos_v[pl.ds(off, L)] = scores[0]
                for kk in range(K):
                    neg_v[kk][pl.ds(off, L)] = scores[1 + kk]
                return 0

            lax.fori_loop(0, GROUPS, group_body, 0)
            return 0

        lax.fori_loop(0, N_CHUNKS, chunk_body, 0)

        pltpu.sync_copy(pos_v, pos_out.at[pl.ds(base_w, N_PER_W)])
        for kk in range(K):
            pltpu.sync_copy(neg_v[kk], neg_out.at[pl.ds(kk * B + base_w, N_PER_W)])

    return sc_scores


_sc_scores = _make_sc_scores()


def _tc_loss_body(pos_ref, neg_ref, out_ref):
    p = pos_ref[...]
    n = neg_ref[...]
    lp = jnp.minimum(p, 0.0) - jnp.log(1.0 + jnp.exp(-jnp.abs(p)))
    ln = jnp.minimum(-n, 0.0) - jnp.log(1.0 + jnp.exp(-jnp.abs(n)))
    out_ref[0, 0] = -(jnp.sum(lp) + jnp.sum(ln)) / B


_tc_loss = pl.pallas_call(
    _tc_loss_body,
    out_shape=jax.ShapeDtypeStruct((1, 1), jnp.float32),
    out_specs=pl.BlockSpec(memory_space=pltpu.SMEM),
)


def kernel(center_nodes, context_nodes, negative_nodes, u_weight, v_weight):
    center = center_nodes.astype(jnp.int32)
    context = context_nodes.astype(jnp.int32)
    neg_t = negative_nodes.astype(jnp.int32).T.reshape(K * B)
    u2 = u_weight.reshape(HALF_ROWS, 2 * D)
    v2 = v_weight.reshape(HALF_ROWS, 2 * D)
    pos, neg = _sc_scores(center, context, neg_t, u2, v2)
    pos2d = pos.reshape(B // 128, 128)
    neg2d = neg.reshape(K * B // 128, 128)
    out = _tc_loss(pos2d, neg2d)
    return out[0, 0]


# speedup vs baseline: 1.6073x; 1.6073x over previous
import functools

import jax
import jax.numpy as jnp
from jax import lax
from jax.experimental import pallas as pl
from jax.experimental.pallas import tpu as pltpu
from jax.experimental.pallas import tpu_sc as plsc

B = 16384
D = 64
K = 5
NC = 2
NS = 16
NW = NC * NS
N_PER_W = B // NW
CHUNK = 64
N_CHUNKS = N_PER_W // CHUNK
L = 16
GROUPS = CHUNK // L


def _make_sc_scores():
    mesh = plsc.VectorSubcoreMesh(core_axis_name="c", subcore_axis_name="s")
    scratch = (
        [pltpu.VMEM((CHUNK,), jnp.int32) for _ in range(2 + K)]
        + [pltpu.VMEM((CHUNK, 2 * D), jnp.float32) for _ in range(2 + K)]
        + [pltpu.VMEM((N_PER_W,), jnp.float32) for _ in range(1 + K)]
        + [pltpu.SemaphoreType.DMA]
    )

    @functools.partial(
        pl.kernel,
        out_type=[
            jax.ShapeDtypeStruct((B,), jnp.float32),
            jax.ShapeDtypeStruct((K * B,), jnp.float32),
        ],
        mesh=mesh,
        scratch_types=scratch,
        compiler_params=pltpu.CompilerParams(
            needs_layout_passes=False, use_tc_tiling_on_sc=True),
    )
    def sc_scores(center_hbm, context_hbm, neg_hbm, w_hbm,
                  pos_out, neg_out,
                  idx_c, idx_x, idx_n0, idx_n1, idx_n2, idx_n3, idx_n4,
                  c_rows, x_rows, n0, n1, n2, n3, n4,
                  pos_v, nv0, nv1, nv2, nv3, nv4, sem):
        idx = [idx_c, idx_x, idx_n0, idx_n1, idx_n2, idx_n3, idx_n4]
        rows = [c_rows, x_rows, n0, n1, n2, n3, n4]
        neg_v = [nv0, nv1, nv2, nv3, nv4]
        wid = lax.axis_index("s") * NC + lax.axis_index("c")
        base_w = wid * N_PER_W
        lanes = lax.iota(jnp.int32, L)

        def chunk_body(ci, _):
            base = base_w + ci * CHUNK
            srcs = [center_hbm, context_hbm] + [neg_hbm] * K
            offs = [base, base] + [kk * B + base for kk in range(K)]
            for t in range(2 + K):
                pltpu.sync_copy(srcs[t].at[pl.ds(offs[t], CHUNK)], idx[t])
            copies = [pltpu.async_copy(w_hbm.at[idx[t]], rows[t], sem)
                      for t in range(2 + K)]
            for cp in copies:
                cp.wait()

            def group_body(g, _, ci=ci):
                goff = g * L
                scores = [jnp.zeros((L,), jnp.float32) for _ in range(1 + K)]
                for i in range(L):
                    e = goff + i
                    sel = lanes == i
                    c4 = [c_rows[e, pl.ds(q * L, L)] for q in range(D // L)]
                    x4 = [x_rows[e, pl.ds(D + q * L, L)] for q in range(D // L)]
                    acc = c4[0] * x4[0]
                    for q in range(1, D // L):
                        acc = acc + c4[q] * x4[q]
                    scores[0] = jnp.where(sel, jnp.sum(acc), scores[0])
                    for kk in range(K):
                        n4 = [rows[2 + kk][e, pl.ds(D + q * L, L)]
                              for q in range(D // L)]
                        acc = c4[0] * n4[0]
                        for q in range(1, D // L):
                            acc = acc + c4[q] * n4[q]
                        scores[1 + kk] = jnp.where(sel, jnp.sum(acc), scores[1 + kk])
                off = ci * CHUNK + goff
                pos_v[pl.ds(off, L)] = scores[0]
                for kk in range(K):
                    neg_v[kk][pl.ds(off, L)] = scores[1 + kk]
                return 0

            lax.fori_loop(0, GROUPS, group_body, 0)
            return 0

        lax.fori_loop(0, N_CHUNKS, chunk_body, 0)

        pltpu.sync_copy(pos_v, pos_out.at[pl.ds(base_w, N_PER_W)])
        for kk in range(K):
            pltpu.sync_copy(neg_v[kk], neg_out.at[pl.ds(kk * B + base_w, N_PER_W)])

    return sc_scores


_sc_scores = _make_sc_scores()


_BLK = 2048
_N_NODES = 1000000


def _build_body(ut_ref, vt_ref, w_ref):
    r = lax.broadcasted_iota(jnp.int32, (D, D), 0)
    c = lax.broadcasted_iota(jnp.int32, (D, D), 1)
    eye = (r == c).astype(jnp.float32)
    dn = (((0,), (0,)), ((), ()))
    w_ref[:, 0:D] = lax.dot_general(
        ut_ref[...], eye, dn, preferred_element_type=jnp.float32)
    w_ref[:, D:2 * D] = lax.dot_general(
        vt_ref[...], eye, dn, preferred_element_type=jnp.float32)


_build_w = pl.pallas_call(
    _build_body,
    grid=((_N_NODES + _BLK - 1) // _BLK,),
    in_specs=[
        pl.BlockSpec((D, _BLK), lambda i: (0, i)),
        pl.BlockSpec((D, _BLK), lambda i: (0, i)),
    ],
    out_specs=pl.BlockSpec((_BLK, 2 * D), lambda i: (i, 0)),
    out_shape=jax.ShapeDtypeStruct((_N_NODES, 2 * D), jnp.float32),
)


def _tc_loss_body(pos_ref, neg_ref, out_ref):
    p = pos_ref[...]
    n = neg_ref[...]
    lp = jnp.minimum(p, 0.0) - jnp.log(1.0 + jnp.exp(-jnp.abs(p)))
    ln = jnp.minimum(-n, 0.0) - jnp.log(1.0 + jnp.exp(-jnp.abs(n)))
    out_ref[0, 0] = -(jnp.sum(lp) + jnp.sum(ln)) / B


_tc_loss = pl.pallas_call(
    _tc_loss_body,
    out_shape=jax.ShapeDtypeStruct((1, 1), jnp.float32),
    out_specs=pl.BlockSpec(memory_space=pltpu.SMEM),
)


def kernel(center_nodes, context_nodes, negative_nodes, u_weight, v_weight):
    center = center_nodes.astype(jnp.int32)
    context = context_nodes.astype(jnp.int32)
    neg_t = negative_nodes.astype(jnp.int32).T.reshape(K * B)
    w = _build_w(u_weight.T, v_weight.T)
    pos, neg = _sc_scores(center, context, neg_t, w)
    pos2d = pos.reshape(B // 128, 128)
    neg2d = neg.reshape(K * B // 128, 128)
    out = _tc_loss(pos2d, neg2d)
    return out[0, 0]


# speedup vs baseline: 1.6683x; 1.0379x over previous
import functools

import jax
import jax.numpy as jnp
from jax import lax
from jax.experimental import pallas as pl
from jax.experimental.pallas import tpu as pltpu
from jax.experimental.pallas import tpu_sc as plsc

B = 16384
D = 64
K = 5
NC = 2
NS = 16
NW = NC * NS
N_PER_W = B // NW
CHUNK = 64
N_CHUNKS = N_PER_W // CHUNK
L = 16
GROUPS = CHUNK // L
N_NODES = 1000000
_BLK = 4096
SPLIT = 123 * _BLK
WPN = D // 2


def _make_sc_scores():
    mesh = plsc.VectorSubcoreMesh(core_axis_name="c", subcore_axis_name="s")
    scratch = (
        [pltpu.VMEM((CHUNK,), jnp.int32) for _ in range(2 + K)]
        + [pltpu.VMEM((CHUNK,), jnp.int32) for _ in range(2 + K)]
        + [pltpu.VMEM((CHUNK, 2 * D), jnp.int32) for _ in range(2 + K)]
        + [pltpu.VMEM((N_PER_W,), jnp.float32) for _ in range(1 + K)]
        + [pltpu.SemaphoreType.DMA]
    )

    @functools.partial(
        pl.kernel,
        out_type=[
            jax.ShapeDtypeStruct((B,), jnp.float32),
            jax.ShapeDtypeStruct((K * B,), jnp.float32),
        ],
        mesh=mesh,
        scratch_types=scratch,
        compiler_params=pltpu.CompilerParams(
            needs_layout_passes=False, use_tc_tiling_on_sc=True),
    )
    def sc_scores(center_hbm, context_hbm, neg_hbm, w_hbm,
                  pos_out, neg_out,
                  idx_c, idx_x, idx_n0, idx_n1, idx_n2, idx_n3, idx_n4,
                  lidx_c, lidx_x, lidx_n0, lidx_n1, lidx_n2, lidx_n3, lidx_n4,
                  c_rows, x_rows, n0, n1, n2, n3, n4,
                  pos_v, nv0, nv1, nv2, nv3, nv4, sem):
        idx_raw = [idx_c, idx_x, idx_n0, idx_n1, idx_n2, idx_n3, idx_n4]
        idx_l = [lidx_c, lidx_x, lidx_n0, lidx_n1, lidx_n2, lidx_n3, lidx_n4]
        rows = [c_rows, x_rows, n0, n1, n2, n3, n4]
        neg_v = [nv0, nv1, nv2, nv3, nv4]
        wid = lax.axis_index("s") * NC + lax.axis_index("c")
        base_w = wid * N_PER_W
        lanes = lax.iota(jnp.int32, L)

        def chunk_body(ci, _):
            base = base_w + ci * CHUNK
            srcs = [center_hbm, context_hbm] + [neg_hbm] * K
            offs = [base, base] + [kk * B + base for kk in range(K)]
            for t in range(2 + K):
                pltpu.sync_copy(srcs[t].at[pl.ds(offs[t], CHUNK)], idx_raw[t])
            for t in range(2 + K):
                for g in range(GROUPS):
                    sl = pl.ds(g * L, L)
                    v = idx_raw[t][sl]
                    idx_l[t][sl] = jnp.where(v >= SPLIT, v - SPLIT, v)
            copies = [pltpu.async_copy(w_hbm.at[idx_l[t]], rows[t], sem)
                      for t in range(2 + K)]
            for cp in copies:
                cp.wait()

            def load_packed(ref, e, off, half):
                out = []
                for q in range(2):
                    w32 = ref[e, pl.ds(off + half * WPN + q * L, L)]
                    out.append(plsc.bitcast(w32 << 16, jnp.float32))
                    out.append(plsc.bitcast(w32 & (-65536), jnp.float32))
                return out

            def group_body(g, _, ci=ci):
                goff = g * L
                pvs = [jnp.where(idx_raw[t][pl.ds(goff, L)] >= SPLIT, D, 0)
                       for t in range(2 + K)]
                scores = [jnp.zeros((L,), jnp.float32) for _ in range(1 + K)]
                for i in range(L):
                    e = goff + i
                    sel = lanes == i
                    c4 = load_packed(c_rows, e, pvs[0][i], 0)
                    x4 = load_packed(x_rows, e, pvs[1][i], 1)
                    acc = c4[0] * x4[0]
                    for q in range(1, 4):
                        acc = acc + c4[q] * x4[q]
                    scores[0] = jnp.where(sel, jnp.sum(acc), scores[0])
                    for kk in range(K):
                        n4 = load_packed(rows[2 + kk], e, pvs[2 + kk][i], 1)
                        acc = c4[0] * n4[0]
                        for q in range(1, 4):
                            acc = acc + c4[q] * n4[q]
                        scores[1 + kk] = jnp.where(sel, jnp.sum(acc), scores[1 + kk])
                off = ci * CHUNK + goff
                pos_v[pl.ds(off, L)] = scores[0]
                for kk in range(K):
                    neg_v[kk][pl.ds(off, L)] = scores[1 + kk]
                return 0

            lax.fori_loop(0, GROUPS, group_body, 0)
            return 0

        lax.fori_loop(0, N_CHUNKS, chunk_body, 0)

        pltpu.sync_copy(pos_v, pos_out.at[pl.ds(base_w, N_PER_W)])
        for kk in range(K):
            pltpu.sync_copy(neg_v[kk], neg_out.at[pl.ds(kk * B + base_w, N_PER_W)])

    return sc_scores


_sc_scores = _make_sc_scores()

def _pack_bf16_pair(m_f32):
    u = lax.bitcast_convert_type(m_f32, jnp.uint32)
    lo = u[:, 0:WPN] >> jnp.uint32(16)
    hi = u[:, WPN:D] & jnp.uint32(0xFFFF0000)
    return lax.bitcast_convert_type(lo | hi, jnp.int32)


def _build_body(ut1_ref, ut2_ref, vt1_ref, vt2_ref, w_ref):
    r = lax.broadcasted_iota(jnp.int32, (D, D), 0)
    c = lax.broadcasted_iota(jnp.int32, (D, D), 1)
    eye = (r == c).astype(jnp.float32)
    dn = (((0,), (0,)), ((), ()))

    def tr(ref):
        return lax.dot_general(ref[...], eye, dn,
                               preferred_element_type=jnp.float32)

    w_ref[:, 0 * WPN:1 * WPN] = _pack_bf16_pair(tr(ut1_ref))
    w_ref[:, 1 * WPN:2 * WPN] = _pack_bf16_pair(tr(vt1_ref))
    w_ref[:, 2 * WPN:3 * WPN] = _pack_bf16_pair(tr(ut2_ref))
    w_ref[:, 3 * WPN:4 * WPN] = _pack_bf16_pair(tr(vt2_ref))


_build_w = pl.pallas_call(
    _build_body,
    grid=(SPLIT // _BLK,),
    in_specs=[
        pl.BlockSpec((D, _BLK), lambda i: (0, i)),
        pl.BlockSpec((D, _BLK),
                     lambda i: (0, jnp.minimum(i + SPLIT // _BLK,
                                               N_NODES // _BLK))),
        pl.BlockSpec((D, _BLK), lambda i: (0, i)),
        pl.BlockSpec((D, _BLK),
                     lambda i: (0, jnp.minimum(i + SPLIT // _BLK,
                                               N_NODES // _BLK))),
    ],
    out_specs=pl.BlockSpec((_BLK, 2 * D), lambda i: (i, 0)),
    out_shape=jax.ShapeDtypeStruct((SPLIT, 2 * D), jnp.int32),
)


def _tc_loss_body(pos_ref, neg_ref, out_ref):
    p = pos_ref[...]
    n = neg_ref[...]
    lp = jnp.minimum(p, 0.0) - jnp.log(1.0 + jnp.exp(-jnp.abs(p)))
    ln = jnp.minimum(-n, 0.0) - jnp.log(1.0 + jnp.exp(-jnp.abs(n)))
    out_ref[0, 0] = -(jnp.sum(lp) + jnp.sum(ln)) / B


_tc_loss = pl.pallas_call(
    _tc_loss_body,
    out_shape=jax.ShapeDtypeStruct((1, 1), jnp.float32),
    out_specs=pl.BlockSpec(memory_space=pltpu.SMEM),
)


def kernel(center_nodes, context_nodes, negative_nodes, u_weight, v_weight):
    center = center_nodes.astype(jnp.int32)
    context = context_nodes.astype(jnp.int32)
    neg_t = negative_nodes.astype(jnp.int32).T.reshape(K * B)
    ut = u_weight.T
    vt = v_weight.T
    w = _build_w(ut, ut, vt, vt)
    pos, neg = _sc_scores(center, context, neg_t, w)
    pos2d = pos.reshape(B // 128, 128)
    neg2d = neg.reshape(K * B // 128, 128)
    out = _tc_loss(pos2d, neg2d)
    return out[0, 0]


# speedup vs baseline: 2.0544x; 1.2315x over previous
import functools

import jax
import jax.numpy as jnp
from jax import lax
from jax.experimental import pallas as pl
from jax.experimental.pallas import tpu as pltpu
from jax.experimental.pallas import tpu_sc as plsc

B = 16384
D = 64
K = 5
NC = 2
NS = 16
NW = NC * NS
N_PER_W = B // NW
CHUNK = 64
N_CHUNKS = N_PER_W // CHUNK
L = 16
GROUPS = CHUNK // L
N_NODES = 1000000
_BLK = 8192
SPLIT = 62 * _BLK
WPN = D // 2


def _make_sc_scores():
    mesh = plsc.VectorSubcoreMesh(core_axis_name="c", subcore_axis_name="s")
    scratch = (
        [pltpu.VMEM((CHUNK,), jnp.int32) for _ in range(2 + K)]
        + [pltpu.VMEM((CHUNK,), jnp.int32) for _ in range(2 + K)]
        + [pltpu.VMEM((CHUNK, 2 * D), jnp.int32) for _ in range(2 + K)]
        + [pltpu.VMEM((N_PER_W,), jnp.float32) for _ in range(1 + K)]
        + [pltpu.SemaphoreType.DMA]
    )

    @functools.partial(
        pl.kernel,
        out_type=[
            jax.ShapeDtypeStruct((B,), jnp.float32),
            jax.ShapeDtypeStruct((K * B,), jnp.float32),
        ],
        mesh=mesh,
        scratch_types=scratch,
        compiler_params=pltpu.CompilerParams(
            needs_layout_passes=False, use_tc_tiling_on_sc=True),
    )
    def sc_scores(center_hbm, context_hbm, neg_hbm, w_hbm,
                  pos_out, neg_out,
                  idx_c, idx_x, idx_n0, idx_n1, idx_n2, idx_n3, idx_n4,
                  lidx_c, lidx_x, lidx_n0, lidx_n1, lidx_n2, lidx_n3, lidx_n4,
                  c_rows, x_rows, n0, n1, n2, n3, n4,
                  pos_v, nv0, nv1, nv2, nv3, nv4, sem):
        idx_raw = [idx_c, idx_x, idx_n0, idx_n1, idx_n2, idx_n3, idx_n4]
        idx_l = [lidx_c, lidx_x, lidx_n0, lidx_n1, lidx_n2, lidx_n3, lidx_n4]
        rows = [c_rows, x_rows, n0, n1, n2, n3, n4]
        neg_v = [nv0, nv1, nv2, nv3, nv4]
        wid = lax.axis_index("s") * NC + lax.axis_index("c")
        base_w = wid * N_PER_W
        lanes = lax.iota(jnp.int32, L)

        def chunk_body(ci, _):
            base = base_w + ci * CHUNK
            srcs = [center_hbm, context_hbm] + [neg_hbm] * K
            offs = [base, base] + [kk * B + base for kk in range(K)]
            for t in range(2 + K):
                pltpu.sync_copy(srcs[t].at[pl.ds(offs[t], CHUNK)], idx_raw[t])
            for t in range(2 + K):
                for g in range(GROUPS):
                    sl = pl.ds(g * L, L)
                    v = idx_raw[t][sl]
                    idx_l[t][sl] = jnp.where(v >= SPLIT, v - SPLIT, v)
            copies = [pltpu.async_copy(w_hbm.at[idx_l[t]], rows[t], sem)
                      for t in range(2 + K)]
            for cp in copies:
                cp.wait()

            def load_packed(ref, e, off, half):
                out = []
                for q in range(2):
                    w32 = ref[e, pl.ds(off + half * WPN + q * L, L)]
                    out.append(plsc.bitcast(w32 << 16, jnp.float32))
                    out.append(plsc.bitcast(w32 & (-65536), jnp.float32))
                return out

            def group_body(g, _, ci=ci):
                goff = g * L
                pvs = [jnp.where(idx_raw[t][pl.ds(goff, L)] >= SPLIT, D, 0)
                       for t in range(2 + K)]
                scores = [jnp.zeros((L,), jnp.float32) for _ in range(1 + K)]
                for i in range(L):
                    e = goff + i
                    sel = lanes == i
                    c4 = load_packed(c_rows, e, pvs[0][i], 0)
                    x4 = load_packed(x_rows, e, pvs[1][i], 1)
                    acc = c4[0] * x4[0]
                    for q in range(1, 4):
                        acc = acc + c4[q] * x4[q]
                    scores[0] = jnp.where(sel, jnp.sum(acc), scores[0])
                    for kk in range(K):
                        n4 = load_packed(rows[2 + kk], e, pvs[2 + kk][i], 1)
                        acc = c4[0] * n4[0]
                        for q in range(1, 4):
                            acc = acc + c4[q] * n4[q]
                        scores[1 + kk] = jnp.where(sel, jnp.sum(acc), scores[1 + kk])
                off = ci * CHUNK + goff
                pos_v[pl.ds(off, L)] = scores[0]
                for kk in range(K):
                    neg_v[kk][pl.ds(off, L)] = scores[1 + kk]
                return 0

            lax.fori_loop(0, GROUPS, group_body, 0)
            return 0

        lax.fori_loop(0, N_CHUNKS, chunk_body, 0)

        pltpu.sync_copy(pos_v, pos_out.at[pl.ds(base_w, N_PER_W)])
        for kk in range(K):
            pltpu.sync_copy(neg_v[kk], neg_out.at[pl.ds(kk * B + base_w, N_PER_W)])

    return sc_scores


_sc_scores = _make_sc_scores()

def _pack_bf16_pair(m_f32):
    u = lax.bitcast_convert_type(m_f32, jnp.uint32)
    lo = u[:, 0:WPN] >> jnp.uint32(16)
    hi = u[:, WPN:D] & jnp.uint32(0xFFFF0000)
    return lax.bitcast_convert_type(lo | hi, jnp.int32)


def _build_body(ut1_ref, ut2_ref, vt1_ref, vt2_ref, w_ref):
    r = lax.broadcasted_iota(jnp.int32, (D, D), 0)
    c = lax.broadcasted_iota(jnp.int32, (D, D), 1)
    eye = (r == c).astype(jnp.bfloat16)
    dn = (((0,), (0,)), ((), ()))

    def tr(ref):
        return lax.dot_general(ref[...].astype(jnp.bfloat16), eye, dn,
                               preferred_element_type=jnp.float32)

    w_ref[:, 0 * WPN:1 * WPN] = _pack_bf16_pair(tr(ut1_ref))
    w_ref[:, 1 * WPN:2 * WPN] = _pack_bf16_pair(tr(vt1_ref))
    w_ref[:, 2 * WPN:3 * WPN] = _pack_bf16_pair(tr(ut2_ref))
    w_ref[:, 3 * WPN:4 * WPN] = _pack_bf16_pair(tr(vt2_ref))


_build_w = pl.pallas_call(
    _build_body,
    grid=(SPLIT // _BLK,),
    in_specs=[
        pl.BlockSpec((D, _BLK), lambda i: (0, i)),
        pl.BlockSpec((D, _BLK),
                     lambda i: (0, jnp.minimum(i + SPLIT // _BLK,
                                               N_NODES // _BLK))),
        pl.BlockSpec((D, _BLK), lambda i: (0, i)),
        pl.BlockSpec((D, _BLK),
                     lambda i: (0, jnp.minimum(i + SPLIT // _BLK,
                                               N_NODES // _BLK))),
    ],
    out_specs=pl.BlockSpec((_BLK, 2 * D), lambda i: (i, 0)),
    out_shape=jax.ShapeDtypeStruct((SPLIT, 2 * D), jnp.int32),
)


def _tc_loss_body(pos_ref, neg_ref, out_ref):
    p = pos_ref[...]
    n = neg_ref[...]
    lp = jnp.minimum(p, 0.0) - jnp.log(1.0 + jnp.exp(-jnp.abs(p)))
    ln = jnp.minimum(-n, 0.0) - jnp.log(1.0 + jnp.exp(-jnp.abs(n)))
    out_ref[0, 0] = -(jnp.sum(lp) + jnp.sum(ln)) / B


_tc_loss = pl.pallas_call(
    _tc_loss_body,
    out_shape=jax.ShapeDtypeStruct((1, 1), jnp.float32),
    out_specs=pl.BlockSpec(memory_space=pltpu.SMEM),
)


def kernel(center_nodes, context_nodes, negative_nodes, u_weight, v_weight):
    center = center_nodes.astype(jnp.int32)
    context = context_nodes.astype(jnp.int32)
    neg_t = negative_nodes.astype(jnp.int32).T.reshape(K * B)
    ut = u_weight.T
    vt = v_weight.T
    w = _build_w(ut, ut, vt, vt)
    pos, neg = _sc_scores(center, context, neg_t, w)
    pos2d = pos.reshape(B // 128, 128)
    neg2d = neg.reshape(K * B // 128, 128)
    out = _tc_loss(pos2d, neg2d)
    return out[0, 0]


# speedup vs baseline: 3.4743x; 1.6911x over previous
import functools

import jax
import jax.numpy as jnp
from jax import lax
from jax.experimental import pallas as pl
from jax.experimental.pallas import tpu as pltpu
from jax.experimental.pallas import tpu_sc as plsc

B = 16384
D = 64
K = 5
NC = 2
NS = 16
NW = NC * NS
N_PER_W = B // NW
CHUNK = 64
N_CHUNKS = N_PER_W // CHUNK
L = 16
GROUPS = CHUNK // L
N_NODES = 1000000
_BLK = 8192
SPLIT = 62 * _BLK
WPN = D // 2


def _make_sc_scores():
    mesh = plsc.VectorSubcoreMesh(core_axis_name="c", subcore_axis_name="s")
    scratch = (
        [pltpu.VMEM((CHUNK,), jnp.int32) for _ in range(2 + K)]
        + [pltpu.VMEM((CHUNK,), jnp.int32) for _ in range(2 + K)]
        + [pltpu.VMEM((CHUNK, 2 * D), jnp.int32) for _ in range(2 + K)]
        + [pltpu.VMEM((N_PER_W,), jnp.float32) for _ in range(1 + K)]
        + [pltpu.SemaphoreType.DMA]
    )

    @functools.partial(
        pl.kernel,
        out_type=[
            jax.ShapeDtypeStruct((B,), jnp.float32),
            jax.ShapeDtypeStruct((K * B,), jnp.float32),
        ],
        mesh=mesh,
        scratch_types=scratch,
        compiler_params=pltpu.CompilerParams(
            needs_layout_passes=False, use_tc_tiling_on_sc=True),
    )
    def sc_scores(center_hbm, context_hbm, neg_hbm, w_hbm,
                  pos_out, neg_out,
                  idx_c, idx_x, idx_n0, idx_n1, idx_n2, idx_n3, idx_n4,
                  lidx_c, lidx_x, lidx_n0, lidx_n1, lidx_n2, lidx_n3, lidx_n4,
                  c_rows, x_rows, n0, n1, n2, n3, n4,
                  pos_v, nv0, nv1, nv2, nv3, nv4, sem):
        idx_raw = [idx_c, idx_x, idx_n0, idx_n1, idx_n2, idx_n3, idx_n4]
        idx_l = [lidx_c, lidx_x, lidx_n0, lidx_n1, lidx_n2, lidx_n3, lidx_n4]
        rows = [c_rows, x_rows, n0, n1, n2, n3, n4]
        neg_v = [nv0, nv1, nv2, nv3, nv4]
        wid = lax.axis_index("s") * NC + lax.axis_index("c")
        base_w = wid * N_PER_W
        lanes = lax.iota(jnp.int32, L)

        def chunk_body(ci, _):
            base = base_w + ci * CHUNK
            srcs = [center_hbm, context_hbm] + [neg_hbm] * K
            offs = [base, base] + [kk * B + base for kk in range(K)]
            for t in range(2 + K):
                pltpu.sync_copy(srcs[t].at[pl.ds(offs[t], CHUNK)], idx_raw[t])
            for t in range(2 + K):
                for g in range(GROUPS):
                    sl = pl.ds(g * L, L)
                    v = idx_raw[t][sl]
                    idx_l[t][sl] = jnp.where(v >= SPLIT, v - SPLIT, v)
            copies = [pltpu.async_copy(w_hbm.at[idx_l[t]], rows[t], sem)
                      for t in range(2 + K)]
            for cp in copies:
                cp.wait()

            def load_packed(ref, e, sh, half):
                out = []
                for q in range(D // L):
                    w32 = ref[e, pl.ds(half * D + q * L, L)]
                    out.append(plsc.bitcast((w32 << sh) & (-65536), jnp.float32))
                return out

            def group_body(g, _, ci=ci):
                goff = g * L
                pvs = [jnp.where(idx_raw[t][pl.ds(goff, L)] >= SPLIT, 0, 16)
                       for t in range(2 + K)]
                scores = [jnp.zeros((L,), jnp.float32) for _ in range(1 + K)]
                for i in range(L):
                    e = goff + i
                    sel = lanes == i
                    c4 = load_packed(c_rows, e, pvs[0][i], 0)
                    x4 = load_packed(x_rows, e, pvs[1][i], 1)
                    acc = c4[0] * x4[0]
                    for q in range(1, 4):
                        acc = acc + c4[q] * x4[q]
                    scores[0] = jnp.where(sel, jnp.sum(acc), scores[0])
                    for kk in range(K):
                        n4 = load_packed(rows[2 + kk], e, pvs[2 + kk][i], 1)
                        acc = c4[0] * n4[0]
                        for q in range(1, 4):
                            acc = acc + c4[q] * n4[q]
                        scores[1 + kk] = jnp.where(sel, jnp.sum(acc), scores[1 + kk])
                off = ci * CHUNK + goff
                pos_v[pl.ds(off, L)] = scores[0]
                for kk in range(K):
                    neg_v[kk][pl.ds(off, L)] = scores[1 + kk]
                return 0

            lax.fori_loop(0, GROUPS, group_body, 0)
            return 0

        lax.fori_loop(0, N_CHUNKS, chunk_body, 0)

        pltpu.sync_copy(pos_v, pos_out.at[pl.ds(base_w, N_PER_W)])
        for kk in range(K):
            pltpu.sync_copy(neg_v[kk], neg_out.at[pl.ds(kk * B + base_w, N_PER_W)])

    return sc_scores


_sc_scores = _make_sc_scores()

def _build_body(ut1_ref, ut2_ref, vt1_ref, vt2_ref, w_ref):
    r = lax.broadcasted_iota(jnp.int32, (D, 2 * D), 0)
    c = lax.broadcasted_iota(jnp.int32, (D, 2 * D), 1)
    eye_u = (c == r).astype(jnp.bfloat16)
    eye_v = (c == r + D).astype(jnp.bfloat16)
    dn = (((0,), (0,)), ((), ()))

    def tr(ref, eye):
        return lax.dot_general(ref[...].astype(jnp.bfloat16), eye, dn,
                               preferred_element_type=jnp.float32)

    p1 = tr(ut1_ref, eye_u) + tr(vt1_ref, eye_v)
    p2 = tr(ut2_ref, eye_u) + tr(vt2_ref, eye_v)
    b1 = lax.bitcast_convert_type(p1, jnp.uint32)
    b2 = lax.bitcast_convert_type(p2, jnp.uint32)
    packed = (b1 >> jnp.uint32(16)) | (b2 & jnp.uint32(0xFFFF0000))
    w_ref[...] = lax.bitcast_convert_type(packed, jnp.int32)


_build_w = pl.pallas_call(
    _build_body,
    grid=(SPLIT // _BLK,),
    in_specs=[
        pl.BlockSpec((D, _BLK), lambda i: (0, i)),
        pl.BlockSpec((D, _BLK),
                     lambda i: (0, jnp.minimum(i + SPLIT // _BLK,
                                               N_NODES // _BLK))),
        pl.BlockSpec((D, _BLK), lambda i: (0, i)),
        pl.BlockSpec((D, _BLK),
                     lambda i: (0, jnp.minimum(i + SPLIT // _BLK,
                                               N_NODES // _BLK))),
    ],
    out_specs=pl.BlockSpec((_BLK, 2 * D), lambda i: (i, 0)),
    out_shape=jax.ShapeDtypeStruct((SPLIT, 2 * D), jnp.int32),
)


def _tc_loss_body(pos_ref, neg_ref, out_ref):
    p = pos_ref[...]
    n = neg_ref[...]
    lp = jnp.minimum(p, 0.0) - jnp.log(1.0 + jnp.exp(-jnp.abs(p)))
    ln = jnp.minimum(-n, 0.0) - jnp.log(1.0 + jnp.exp(-jnp.abs(n)))
    out_ref[0, 0] = -(jnp.sum(lp) + jnp.sum(ln)) / B


_tc_loss = pl.pallas_call(
    _tc_loss_body,
    out_shape=jax.ShapeDtypeStruct((1, 1), jnp.float32),
    out_specs=pl.BlockSpec(memory_space=pltpu.SMEM),
)


def kernel(center_nodes, context_nodes, negative_nodes, u_weight, v_weight):
    center = center_nodes.astype(jnp.int32)
    context = context_nodes.astype(jnp.int32)
    neg_t = negative_nodes.astype(jnp.int32).T.reshape(K * B)
    ut = u_weight.T
    vt = v_weight.T
    w = _build_w(ut, ut, vt, vt)
    pos, neg = _sc_scores(center, context, neg_t, w)
    pos2d = pos.reshape(B // 128, 128)
    neg2d = neg.reshape(K * B // 128, 128)
    out = _tc_loss(pos2d, neg2d)
    return out[0, 0]


# speedup vs baseline: 3.7737x; 1.0862x over previous
import functools

import jax
import jax.numpy as jnp
from jax import lax
from jax.experimental import pallas as pl
from jax.experimental.pallas import tpu as pltpu
from jax.experimental.pallas import tpu_sc as plsc

B = 16384
D = 64
K = 5
NC = 2
NS = 16
NW = NC * NS
N_PER_W = B // NW
CHUNK = 128
N_CHUNKS = N_PER_W // CHUNK
L = 16
GROUPS = CHUNK // L
N_NODES = 1000000
_BLK = 8192
SPLIT = 62 * _BLK
WPN = D // 2


def _make_sc_scores():
    mesh = plsc.VectorSubcoreMesh(core_axis_name="c", subcore_axis_name="s")
    scratch = (
        [pltpu.VMEM((CHUNK,), jnp.int32) for _ in range(2 + K)]
        + [pltpu.VMEM((CHUNK,), jnp.int32) for _ in range(2 + K)]
        + [pltpu.VMEM((CHUNK, 2 * D), jnp.int32) for _ in range(2 + K)]
        + [pltpu.VMEM((N_PER_W,), jnp.float32) for _ in range(1 + K)]
        + [pltpu.SemaphoreType.DMA]
    )

    @functools.partial(
        pl.kernel,
        out_type=[
            jax.ShapeDtypeStruct((B,), jnp.float32),
            jax.ShapeDtypeStruct((K * B,), jnp.float32),
        ],
        mesh=mesh,
        scratch_types=scratch,
        compiler_params=pltpu.CompilerParams(
            needs_layout_passes=False, use_tc_tiling_on_sc=True),
    )
    def sc_scores(center_hbm, context_hbm, neg_hbm, w_hbm,
                  pos_out, neg_out,
                  idx_c, idx_x, idx_n0, idx_n1, idx_n2, idx_n3, idx_n4,
                  lidx_c, lidx_x, lidx_n0, lidx_n1, lidx_n2, lidx_n3, lidx_n4,
                  c_rows, x_rows, n0, n1, n2, n3, n4,
                  pos_v, nv0, nv1, nv2, nv3, nv4, sem):
        idx_raw = [idx_c, idx_x, idx_n0, idx_n1, idx_n2, idx_n3, idx_n4]
        idx_l = [lidx_c, lidx_x, lidx_n0, lidx_n1, lidx_n2, lidx_n3, lidx_n4]
        rows = [c_rows, x_rows, n0, n1, n2, n3, n4]
        neg_v = [nv0, nv1, nv2, nv3, nv4]
        wid = lax.axis_index("s") * NC + lax.axis_index("c")
        base_w = wid * N_PER_W
        lanes = lax.iota(jnp.int32, L)

        def chunk_body(ci, _):
            base = base_w + ci * CHUNK
            srcs = [center_hbm, context_hbm] + [neg_hbm] * K
            offs = [base, base] + [kk * B + base for kk in range(K)]
            idx_copies = [
                pltpu.async_copy(srcs[t].at[pl.ds(offs[t], CHUNK)],
                                 idx_raw[t], sem)
                for t in range(2 + K)]
            for cp in idx_copies:
                cp.wait()
            for t in range(2 + K):
                for g in range(GROUPS):
                    sl = pl.ds(g * L, L)
                    v = idx_raw[t][sl]
                    idx_l[t][sl] = jnp.where(v >= SPLIT, v - SPLIT, v)
            copies = [pltpu.async_copy(w_hbm.at[idx_l[t]], rows[t], sem)
                      for t in range(2 + K)]
            for cp in copies:
                cp.wait()

            def load_packed(ref, e, sh, half):
                out = []
                for q in range(D // L):
                    w32 = ref[e, pl.ds(half * D + q * L, L)]
                    out.append(plsc.bitcast((w32 << sh) & (-65536), jnp.float32))
                return out

            def group_body(g, _, ci=ci):
                goff = g * L
                pvs = [jnp.where(idx_raw[t][pl.ds(goff, L)] >= SPLIT, 0, 16)
                       for t in range(2 + K)]
                scores = [jnp.zeros((L,), jnp.float32) for _ in range(1 + K)]
                for i in range(L):
                    e = goff + i
                    sel = lanes == i
                    c4 = load_packed(c_rows, e, pvs[0][i], 0)
                    x4 = load_packed(x_rows, e, pvs[1][i], 1)
                    acc = c4[0] * x4[0]
                    for q in range(1, 4):
                        acc = acc + c4[q] * x4[q]
                    scores[0] = jnp.where(sel, jnp.sum(acc), scores[0])
                    for kk in range(K):
                        n4 = load_packed(rows[2 + kk], e, pvs[2 + kk][i], 1)
                        acc = c4[0] * n4[0]
                        for q in range(1, 4):
                            acc = acc + c4[q] * n4[q]
                        scores[1 + kk] = jnp.where(sel, jnp.sum(acc), scores[1 + kk])
                off = ci * CHUNK + goff
                pos_v[pl.ds(off, L)] = scores[0]
                for kk in range(K):
                    neg_v[kk][pl.ds(off, L)] = scores[1 + kk]
                return 0

            lax.fori_loop(0, GROUPS, group_body, 0)
            return 0

        lax.fori_loop(0, N_CHUNKS, chunk_body, 0)

        pltpu.sync_copy(pos_v, pos_out.at[pl.ds(base_w, N_PER_W)])
        for kk in range(K):
            pltpu.sync_copy(neg_v[kk], neg_out.at[pl.ds(kk * B + base_w, N_PER_W)])

    return sc_scores


_sc_scores = _make_sc_scores()

def _build_body(ut1_ref, ut2_ref, vt1_ref, vt2_ref, w_ref):
    r = lax.broadcasted_iota(jnp.int32, (D, 2 * D), 0)
    c = lax.broadcasted_iota(jnp.int32, (D, 2 * D), 1)
    eye_u = (c == r).astype(jnp.bfloat16)
    eye_v = (c == r + D).astype(jnp.bfloat16)
    dn = (((0,), (0,)), ((), ()))

    def tr(ref, eye):
        return lax.dot_general(ref[...].astype(jnp.bfloat16), eye, dn,
                               preferred_element_type=jnp.float32)

    p1 = tr(ut1_ref, eye_u) + tr(vt1_ref, eye_v)
    p2 = tr(ut2_ref, eye_u) + tr(vt2_ref, eye_v)
    b1 = lax.bitcast_convert_type(p1, jnp.uint32)
    b2 = lax.bitcast_convert_type(p2, jnp.uint32)
    packed = (b1 >> jnp.uint32(16)) | (b2 & jnp.uint32(0xFFFF0000))
    w_ref[...] = lax.bitcast_convert_type(packed, jnp.int32)


_build_w = pl.pallas_call(
    _build_body,
    grid=(SPLIT // _BLK,),
    in_specs=[
        pl.BlockSpec((D, _BLK), lambda i: (0, i)),
        pl.BlockSpec((D, _BLK),
                     lambda i: (0, jnp.minimum(i + SPLIT // _BLK,
                                               N_NODES // _BLK))),
        pl.BlockSpec((D, _BLK), lambda i: (0, i)),
        pl.BlockSpec((D, _BLK),
                     lambda i: (0, jnp.minimum(i + SPLIT // _BLK,
                                               N_NODES // _BLK))),
    ],
    out_specs=pl.BlockSpec((_BLK, 2 * D), lambda i: (i, 0)),
    out_shape=jax.ShapeDtypeStruct((SPLIT, 2 * D), jnp.int32),
)


def _tc_loss_body(pos_ref, neg_ref, out_ref):
    p = pos_ref[...]
    n = neg_ref[...]
    lp = jnp.minimum(p, 0.0) - jnp.log(1.0 + jnp.exp(-jnp.abs(p)))
    ln = jnp.minimum(-n, 0.0) - jnp.log(1.0 + jnp.exp(-jnp.abs(n)))
    out_ref[0, 0] = -(jnp.sum(lp) + jnp.sum(ln)) / B


_tc_loss = pl.pallas_call(
    _tc_loss_body,
    out_shape=jax.ShapeDtypeStruct((1, 1), jnp.float32),
    out_specs=pl.BlockSpec(memory_space=pltpu.SMEM),
)


def kernel(center_nodes, context_nodes, negative_nodes, u_weight, v_weight):
    center = center_nodes.astype(jnp.int32)
    context = context_nodes.astype(jnp.int32)
    neg_t = negative_nodes.astype(jnp.int32).T.reshape(K * B)
    ut = u_weight.T
    vt = v_weight.T
    w = _build_w(ut, ut, vt, vt)
    pos, neg = _sc_scores(center, context, neg_t, w)
    pos2d = pos.reshape(B // 128, 128)
    neg2d = neg.reshape(K * B // 128, 128)
    out = _tc_loss(pos2d, neg2d)
    return out[0, 0]


# speedup vs baseline: 3.8845x; 1.0294x over previous
import functools

import jax
import jax.numpy as jnp
from jax import lax
from jax.experimental import pallas as pl
from jax.experimental.pallas import tpu as pltpu
from jax.experimental.pallas import tpu_sc as plsc

B = 16384
D = 64
K = 5
NC = 2
NS = 16
NW = NC * NS
N_PER_W = B // NW
CHUNK = 64
N_CHUNKS = N_PER_W // CHUNK
L = 16
GROUPS = CHUNK // L
N_NODES = 1000000
_BLK = 8192
SPLIT = 62 * _BLK
WPN = D // 2


def _make_sc_scores():
    mesh = plsc.VectorSubcoreMesh(core_axis_name="c", subcore_axis_name="s")
    scratch = (
        [pltpu.VMEM((2, CHUNK), jnp.int32) for _ in range(2 + K)]
        + [pltpu.VMEM((2, CHUNK), jnp.int32) for _ in range(2 + K)]
        + [pltpu.VMEM((2, CHUNK, 2 * D), jnp.int32) for _ in range(2 + K)]
        + [pltpu.VMEM((N_PER_W,), jnp.float32) for _ in range(1 + K)]
        + [pltpu.SemaphoreType.DMA((2,)), pltpu.SemaphoreType.DMA((2,))]
    )

    @functools.partial(
        pl.kernel,
        out_type=[
            jax.ShapeDtypeStruct((B,), jnp.float32),
            jax.ShapeDtypeStruct((K * B,), jnp.float32),
        ],
        mesh=mesh,
        scratch_types=scratch,
        compiler_params=pltpu.CompilerParams(
            needs_layout_passes=False, use_tc_tiling_on_sc=True),
    )
    def sc_scores(center_hbm, context_hbm, neg_hbm, w_hbm,
                  pos_out, neg_out,
                  idx_c, idx_x, idx_n0, idx_n1, idx_n2, idx_n3, idx_n4,
                  lidx_c, lidx_x, lidx_n0, lidx_n1, lidx_n2, lidx_n3, lidx_n4,
                  c_rows, x_rows, n0, n1, n2, n3, n4,
                  pos_v, nv0, nv1, nv2, nv3, nv4, sem_i, sem_r):
        idx_raw = [idx_c, idx_x, idx_n0, idx_n1, idx_n2, idx_n3, idx_n4]
        idx_l = [lidx_c, lidx_x, lidx_n0, lidx_n1, lidx_n2, lidx_n3, lidx_n4]
        rows = [c_rows, x_rows, n0, n1, n2, n3, n4]
        neg_v = [nv0, nv1, nv2, nv3, nv4]
        wid = lax.axis_index("s") * NC + lax.axis_index("c")
        base_w = wid * N_PER_W
        lanes = lax.iota(jnp.int32, L)
        srcs = [center_hbm, context_hbm] + [neg_hbm] * K

        def fetch_map_fire(ci, b):
            base = base_w + ci * CHUNK
            offs = [base, base] + [kk * B + base for kk in range(K)]
            idx_copies = [
                pltpu.async_copy(srcs[t].at[pl.ds(offs[t], CHUNK)],
                                 idx_raw[t].at[b], sem_i.at[b])
                for t in range(2 + K)]
            for cp in idx_copies:
                cp.wait()
            for t in range(2 + K):
                for g in range(GROUPS):
                    sl = pl.ds(g * L, L)
                    v = idx_raw[t][b, sl]
                    idx_l[t][b, sl] = jnp.where(v >= SPLIT, v - SPLIT, v)
            for t in range(2 + K):
                pltpu.async_copy(w_hbm.at[idx_l[t].at[b]], rows[t].at[b],
                                 sem_r.at[b])

        fetch_map_fire(0, 0)

        def chunk_body(ci, _):
            b = ci & 1
            fetch_map_fire(jnp.minimum(ci + 1, N_CHUNKS - 1), b ^ 1)
            for t in range(2 + K):
                pltpu.make_async_copy(w_hbm.at[idx_l[t].at[b]],
                                      rows[t].at[b], sem_r.at[b]).wait()

            def load_packed(ref, e, sh, half, b=b):
                out = []
                for q in range(D // L):
                    w32 = ref[b, e, pl.ds(half * D + q * L, L)]
                    out.append(plsc.bitcast((w32 << sh) & (-65536), jnp.float32))
                return out

            def group_body(g, _, ci=ci, b=b):
                goff = g * L
                pvs = [jnp.where(idx_raw[t][b, pl.ds(goff, L)] >= SPLIT, 0, 16)
                       for t in range(2 + K)]
                scores = [jnp.zeros((L,), jnp.float32) for _ in range(1 + K)]
                for i in range(L):
                    e = goff + i
                    sel = lanes == i
                    c4 = load_packed(c_rows, e, pvs[0][i], 0)
                    x4 = load_packed(x_rows, e, pvs[1][i], 1)
                    acc = c4[0] * x4[0]
                    for q in range(1, 4):
                        acc = acc + c4[q] * x4[q]
                    scores[0] = jnp.where(sel, jnp.sum(acc), scores[0])
                    for kk in range(K):
                        n4 = load_packed(rows[2 + kk], e, pvs[2 + kk][i], 1)
                        acc = c4[0] * n4[0]
                        for q in range(1, 4):
                            acc = acc + c4[q] * n4[q]
                        scores[1 + kk] = jnp.where(sel, jnp.sum(acc), scores[1 + kk])
                off = ci * CHUNK + goff
                pos_v[pl.ds(off, L)] = scores[0]
                for kk in range(K):
                    neg_v[kk][pl.ds(off, L)] = scores[1 + kk]
                return 0

            lax.fori_loop(0, GROUPS, group_body, 0)
            return 0

        lax.fori_loop(0, N_CHUNKS, chunk_body, 0)

        for t in range(2 + K):
            pltpu.make_async_copy(w_hbm.at[idx_l[t].at[0]], rows[t].at[0],
                                  sem_r.at[0]).wait()

        pltpu.sync_copy(pos_v, pos_out.at[pl.ds(base_w, N_PER_W)])
        for kk in range(K):
            pltpu.sync_copy(neg_v[kk], neg_out.at[pl.ds(kk * B + base_w, N_PER_W)])

    return sc_scores


_sc_scores = _make_sc_scores()

def _build_body(ut1_ref, ut2_ref, vt1_ref, vt2_ref, w_ref):
    r = lax.broadcasted_iota(jnp.int32, (D, 2 * D), 0)
    c = lax.broadcasted_iota(jnp.int32, (D, 2 * D), 1)
    eye_u = (c == r).astype(jnp.bfloat16)
    eye_v = (c == r + D).astype(jnp.bfloat16)
    dn = (((0,), (0,)), ((), ()))

    def tr(ref, eye):
        return lax.dot_general(ref[...].astype(jnp.bfloat16), eye, dn,
                               preferred_element_type=jnp.float32)

    p1 = tr(ut1_ref, eye_u) + tr(vt1_ref, eye_v)
    p2 = tr(ut2_ref, eye_u) + tr(vt2_ref, eye_v)
    b1 = lax.bitcast_convert_type(p1, jnp.uint32)
    b2 = lax.bitcast_convert_type(p2, jnp.uint32)
    packed = (b1 >> jnp.uint32(16)) | (b2 & jnp.uint32(0xFFFF0000))
    w_ref[...] = lax.bitcast_convert_type(packed, jnp.int32)


_build_w = pl.pallas_call(
    _build_body,
    grid=(SPLIT // _BLK,),
    in_specs=[
        pl.BlockSpec((D, _BLK), lambda i: (0, i)),
        pl.BlockSpec((D, _BLK),
                     lambda i: (0, jnp.minimum(i + SPLIT // _BLK,
                                               N_NODES // _BLK))),
        pl.BlockSpec((D, _BLK), lambda i: (0, i)),
        pl.BlockSpec((D, _BLK),
                     lambda i: (0, jnp.minimum(i + SPLIT // _BLK,
                                               N_NODES // _BLK))),
    ],
    out_specs=pl.BlockSpec((_BLK, 2 * D), lambda i: (i, 0)),
    out_shape=jax.ShapeDtypeStruct((SPLIT, 2 * D), jnp.int32),
)


def _tc_loss_body(pos_ref, neg_ref, out_ref):
    p = pos_ref[...]
    n = neg_ref[...]
    lp = jnp.minimum(p, 0.0) - jnp.log(1.0 + jnp.exp(-jnp.abs(p)))
    ln = jnp.minimum(-n, 0.0) - jnp.log(1.0 + jnp.exp(-jnp.abs(n)))
    out_ref[0, 0] = -(jnp.sum(lp) + jnp.sum(ln)) / B


_tc_loss = pl.pallas_call(
    _tc_loss_body,
    out_shape=jax.ShapeDtypeStruct((1, 1), jnp.float32),
    out_specs=pl.BlockSpec(memory_space=pltpu.SMEM),
)


def kernel(center_nodes, context_nodes, negative_nodes, u_weight, v_weight):
    center = center_nodes.astype(jnp.int32)
    context = context_nodes.astype(jnp.int32)
    neg_t = negative_nodes.astype(jnp.int32).T.reshape(K * B)
    ut = u_weight.T
    vt = v_weight.T
    w = _build_w(ut, ut, vt, vt)
    pos, neg = _sc_scores(center, context, neg_t, w)
    pos2d = pos.reshape(B // 128, 128)
    neg2d = neg.reshape(K * B // 128, 128)
    out = _tc_loss(pos2d, neg2d)
    return out[0, 0]
